# Initial kernel scaffold; baseline (speedup 1.0000x reference)
import jax
import jax.numpy as jnp
from jax.experimental import pallas as pl


def kernel(hidden_states, W, expert_bias):
    raise NotImplementedError("write your pallas kernel here")


# speedup vs baseline: 4.9505x; 4.9505x over previous
import functools

import jax
import jax.numpy as jnp
from jax.experimental import pallas as pl

NUM_EXPERTS = 64
TOP_K = 8
BALANCE_FACTOR = 1e-4


def _router_body(x_ref, wt_ref, b_ref, dm_ref, idx_ref, load_ref, loss_ref,
                 *, num_tiles, total_tokens):
    i = pl.program_id(0)
    x = x_ref[...]
    logits = jnp.dot(x, wt_ref[...], preferred_element_type=jnp.float32)
    logits = logits + b_ref[...]

    m = jnp.max(logits, axis=-1, keepdims=True)
    e = jnp.exp(logits - m)
    probs = e / jnp.sum(e, axis=-1, keepdims=True)

    cols = jax.lax.broadcasted_iota(jnp.int32, probs.shape, 1)
    work = probs
    sel = jnp.zeros(probs.shape, dtype=jnp.bool_)
    idx_cols = []
    for _ in range(TOP_K):
        mk = jnp.max(work, axis=-1, keepdims=True)
        is_max = work == mk
        amax = jnp.min(jnp.where(is_max, cols, NUM_EXPERTS), axis=-1,
                       keepdims=True)
        idx_cols.append(amax)
        hit = cols == amax
        sel = jnp.logical_or(sel, hit)
        work = jnp.where(hit, -1.0, work)

    idx_ref[...] = jnp.concatenate(idx_cols, axis=1)
    psel = jnp.where(sel, probs, 0.0)
    dm_ref[...] = psel / jnp.sum(psel, axis=-1, keepdims=True)

    part = jnp.sum(probs, axis=0, keepdims=True)

    @pl.when(i == 0)
    def _init():
        load_ref[...] = part

    @pl.when(i > 0)
    def _acc():
        load_ref[...] = load_ref[...] + part

    @pl.when(i == num_tiles - 1)
    def _finish():
        load = load_ref[...] / total_tokens
        target = 1.0 / NUM_EXPERTS
        kl = target * (jnp.log(target) - jnp.log(load))
        loss_ref[...] = jnp.sum(kl, axis=1, keepdims=True) * (
            BALANCE_FACTOR / NUM_EXPERTS)


def kernel(hidden_states, W, expert_bias):
    Bb, Ss, Dd = hidden_states.shape
    T = Bb * Ss
    TT = 512
    num_tiles = T // TT

    x = hidden_states.reshape(T, Dd)
    wt = W.T
    bias = expert_bias.reshape(1, NUM_EXPERTS)

    body = functools.partial(_router_body, num_tiles=num_tiles,
                             total_tokens=float(T))

    dm, idx, _, loss = pl.pallas_call(
        body,
        grid=(num_tiles,),
        in_specs=[
            pl.BlockSpec((TT, Dd), lambda i: (i, 0)),
            pl.BlockSpec((Dd, NUM_EXPERTS), lambda i: (0, 0)),
            pl.BlockSpec((1, NUM_EXPERTS), lambda i: (0, 0)),
        ],
        out_specs=[
            pl.BlockSpec((TT, NUM_EXPERTS), lambda i: (i, 0)),
            pl.BlockSpec((TT, TOP_K), lambda i: (i, 0)),
            pl.BlockSpec((1, NUM_EXPERTS), lambda i: (0, 0)),
            pl.BlockSpec((1, 1), lambda i: (0, 0)),
        ],
        out_shape=[
            jax.ShapeDtypeStruct((T, NUM_EXPERTS), jnp.float32),
            jax.ShapeDtypeStruct((T, TOP_K), jnp.int32),
            jax.ShapeDtypeStruct((1, NUM_EXPERTS), jnp.float32),
            jax.ShapeDtypeStruct((1, 1), jnp.float32),
        ],
    )(x, wt, bias)

    dispatch_mask = dm.reshape(Bb, Ss, NUM_EXPERTS)
    top_k_indices = idx.reshape(Bb, Ss, TOP_K)
    balance_loss = loss.reshape(())
    return dispatch_mask, balance_loss, top_k_indices


# speedup vs baseline: 10.0693x; 2.0340x over previous
import functools

import jax
import jax.numpy as jnp
from jax.experimental import pallas as pl

NUM_EXPERTS = 64
TOP_K = 8
BALANCE_FACTOR = 1e-4


def _router_body(x_ref, w_ref, b_ref, dm_ref, idx_ref, load_ref, loss_ref,
                 *, num_tiles, total_tokens):
    i = pl.program_id(0)
    lt = jax.lax.dot_general(
        w_ref[...], x_ref[...],
        dimension_numbers=(((1,), (1,)), ((), ())),
        preferred_element_type=jnp.float32)
    lt = lt + b_ref[...]

    m = jnp.max(lt, axis=0, keepdims=True)
    e = jnp.exp(lt - m)
    probs = e / jnp.sum(e, axis=0, keepdims=True)

    rows_f = jax.lax.broadcasted_iota(jnp.int32, probs.shape, 0).astype(
        jnp.float32)
    work = probs
    idx_rows = []
    for _ in range(TOP_K):
        mk = jnp.max(work, axis=0, keepdims=True)
        amax = jnp.min(jnp.where(work == mk, rows_f, float(NUM_EXPERTS)),
                       axis=0, keepdims=True)
        idx_rows.append(amax)
        work = jnp.where(rows_f == amax, -1.0, work)

    idx_ref[...] = jnp.concatenate(idx_rows, axis=0).astype(jnp.int32)
    psel = jnp.where(work < 0.0, probs, 0.0)
    dm_ref[...] = psel / jnp.sum(psel, axis=0, keepdims=True)

    part = jnp.sum(probs, axis=1, keepdims=True)

    @pl.when(i == 0)
    def _init():
        load_ref[...] = part

    @pl.when(i > 0)
    def _acc():
        load_ref[...] = load_ref[...] + part

    @pl.when(i == num_tiles - 1)
    def _finish():
        load = load_ref[...] / total_tokens
        target = 1.0 / NUM_EXPERTS
        kl = target * (jnp.log(target) - jnp.log(load))
        loss_ref[...] = jnp.sum(kl, axis=0, keepdims=True) * (
            BALANCE_FACTOR / NUM_EXPERTS)


def kernel(hidden_states, W, expert_bias):
    Bb, Ss, Dd = hidden_states.shape
    T = Bb * Ss
    TT = 1024
    num_tiles = T // TT

    x = hidden_states.reshape(T, Dd)
    bias = expert_bias.reshape(NUM_EXPERTS, 1)

    body = functools.partial(_router_body, num_tiles=num_tiles,
                             total_tokens=float(T))

    dmt, idxt, _, loss = pl.pallas_call(
        body,
        grid=(num_tiles,),
        in_specs=[
            pl.BlockSpec((TT, Dd), lambda i: (i, 0)),
            pl.BlockSpec((NUM_EXPERTS, Dd), lambda i: (0, 0)),
            pl.BlockSpec((NUM_EXPERTS, 1), lambda i: (0, 0)),
        ],
        out_specs=[
            pl.BlockSpec((NUM_EXPERTS, TT), lambda i: (0, i)),
            pl.BlockSpec((TOP_K, TT), lambda i: (0, i)),
            pl.BlockSpec((NUM_EXPERTS, 1), lambda i: (0, 0)),
            pl.BlockSpec((1, 1), lambda i: (0, 0)),
        ],
        out_shape=[
            jax.ShapeDtypeStruct((NUM_EXPERTS, T), jnp.float32),
            jax.ShapeDtypeStruct((TOP_K, T), jnp.int32),
            jax.ShapeDtypeStruct((NUM_EXPERTS, 1), jnp.float32),
            jax.ShapeDtypeStruct((1, 1), jnp.float32),
        ],
    )(x, W, bias)

    dispatch_mask = dmt.T.reshape(Bb, Ss, NUM_EXPERTS)
    top_k_indices = idxt.T.reshape(Bb, Ss, TOP_K)
    balance_loss = loss.reshape(())
    return dispatch_mask, balance_loss, top_k_indices
